# Initial kernel scaffold; baseline (speedup 1.0000x reference)
import functools

import jax
import jax.numpy as jnp
from jax import lax
from jax.experimental import pallas as pl
from jax.experimental.pallas import tpu as pltpu
from jax.experimental.pallas import tpu_sc as plsc

D = 64
B = 16384

_ROWS_PER_BLOCK = 2048


def _sc_gather(table1, table2, idx1, idx2):
    info = plsc.get_sparse_core_info()
    nc, ns = info.num_cores, info.num_subcores
    nw = nc * ns
    b_per_w = B // nw
    mesh = plsc.VectorSubcoreMesh(core_axis_name="c", subcore_axis_name="s")

    @functools.partial(
        pl.kernel,
        mesh=mesh,
        out_type=(
            jax.ShapeDtypeStruct((B, D), jnp.float32),
            jax.ShapeDtypeStruct((B, D), jnp.float32),
        ),
        scratch_types=[
            pltpu.VMEM((b_per_w,), jnp.int32),
            pltpu.VMEM((b_per_w,), jnp.int32),
            pltpu.VMEM((b_per_w, D), jnp.float32),
            pltpu.VMEM((b_per_w, D), jnp.float32),
            pltpu.SemaphoreType.DMA,
            pltpu.SemaphoreType.DMA,
        ],
    )
    def gather_kernel(t1, t2, i1, i2, o1, o2, iv1, iv2, rows1, rows2, s1, s2):
        wid = lax.axis_index("s") * nc + lax.axis_index("c")
        base = wid * b_per_w
        pltpu.sync_copy(i1.at[pl.ds(base, b_per_w)], iv1)
        pltpu.sync_copy(i2.at[pl.ds(base, b_per_w)], iv2)
        c1 = pltpu.async_copy(t1.at[iv1], rows1, s1)
        c2 = pltpu.async_copy(t2.at[iv2], rows2, s2)
        c1.wait()
        c2.wait()
        pltpu.sync_copy(rows1, o1.at[pl.ds(base, b_per_w)])
        pltpu.sync_copy(rows2, o2.at[pl.ds(base, b_per_w)])

    return gather_kernel(table1, table2, idx1, idx2)


def _bind_cosine_body(e1_ref, e2_ref, c_ref, out_ref):
    c = c_ref[...]
    w1 = jnp.dot(e1_ref[...], c, preferred_element_type=jnp.float32)
    w2 = jnp.dot(e2_ref[...], c, preferred_element_type=jnp.float32)
    num = jnp.sum(w1 * w2, axis=-1)
    n1 = jnp.sum(w1 * w1, axis=-1)
    n2 = jnp.sum(w2 * w2, axis=-1)
    out_ref[...] = (num / (jnp.sqrt(n1) * jnp.sqrt(n2) + 1e-8))[None, :]


def _bind_cosine(e1, e2, circ, interpret=False):
    r = _ROWS_PER_BLOCK
    g = B // r
    out = pl.pallas_call(
        _bind_cosine_body,
        grid=(g,),
        in_specs=[
            pl.BlockSpec((r, D), lambda i: (i, 0)),
            pl.BlockSpec((r, D), lambda i: (i, 0)),
            pl.BlockSpec((D, D), lambda i: (0, 0)),
        ],
        out_specs=pl.BlockSpec((1, r), lambda i: (i, 0)),
        out_shape=jax.ShapeDtypeStruct((g, r), jnp.float32),
        interpret=interpret,
    )(e1, e2, circ)
    return out.reshape(B)


def kernel(source1, source2, dummy_vector, word1, word2):
    i1 = word1.astype(jnp.int32)
    i2 = word2.astype(jnp.int32)
    e1, e2 = _sc_gather(source1, source2, i1, i2)
    shift = (jnp.arange(D)[None, :] - jnp.arange(D)[:, None]) % D
    circ = dummy_vector[shift]
    return _bind_cosine(e1, e2, circ)


# speedup vs baseline: 2.2082x; 2.2082x over previous
import functools

import jax
import jax.numpy as jnp
from jax import lax
from jax.experimental import pallas as pl
from jax.experimental.pallas import tpu as pltpu
from jax.experimental.pallas import tpu_sc as plsc

D = 64
B = 16384

_ROWS_PER_BLOCK = 2048


def _sc_gather(table1, table2, idx1, idx2):
    info = plsc.get_sparse_core_info()
    nc, ns = info.num_cores, info.num_subcores
    nw = nc * ns
    b_per_w = B // nw
    mesh = plsc.VectorSubcoreMesh(core_axis_name="c", subcore_axis_name="s")

    @functools.partial(
        pl.kernel,
        mesh=mesh,
        compiler_params=pltpu.CompilerParams(use_tc_tiling_on_sc=False),
        out_type=(
            jax.ShapeDtypeStruct((B, D), jnp.float32),
            jax.ShapeDtypeStruct((B, D), jnp.float32),
        ),
        scratch_types=[
            pltpu.VMEM((b_per_w,), jnp.int32),
            pltpu.VMEM((b_per_w,), jnp.int32),
            pltpu.VMEM((b_per_w, D), jnp.float32),
            pltpu.VMEM((b_per_w, D), jnp.float32),
            pltpu.SemaphoreType.DMA,
            pltpu.SemaphoreType.DMA,
        ],
    )
    def gather_kernel(t1, t2, i1, i2, o1, o2, iv1, iv2, rows1, rows2, s1, s2):
        wid = lax.axis_index("s") * nc + lax.axis_index("c")
        base = wid * b_per_w
        pltpu.sync_copy(i1.at[pl.ds(base, b_per_w)], iv1)
        pltpu.sync_copy(i2.at[pl.ds(base, b_per_w)], iv2)
        c1 = pltpu.async_copy(t1.at[iv1], rows1, s1)
        c2 = pltpu.async_copy(t2.at[iv2], rows2, s2)
        c1.wait()
        c2.wait()
        pltpu.sync_copy(rows1, o1.at[pl.ds(base, b_per_w)])
        pltpu.sync_copy(rows2, o2.at[pl.ds(base, b_per_w)])

    return gather_kernel(table1, table2, idx1, idx2)


def _bind_cosine_body(e1_ref, e2_ref, c_ref, out_ref):
    c = c_ref[...]
    w1 = jnp.dot(e1_ref[...], c, preferred_element_type=jnp.float32)
    w2 = jnp.dot(e2_ref[...], c, preferred_element_type=jnp.float32)
    num = jnp.sum(w1 * w2, axis=-1)
    n1 = jnp.sum(w1 * w1, axis=-1)
    n2 = jnp.sum(w2 * w2, axis=-1)
    out_ref[...] = num / (jnp.sqrt(n1) * jnp.sqrt(n2) + 1e-8)


def _bind_cosine(e1, e2, circ, interpret=False):
    r = _ROWS_PER_BLOCK
    g = B // r
    out = pl.pallas_call(
        _bind_cosine_body,
        grid=(g,),
        in_specs=[
            pl.BlockSpec((r, D), lambda i: (i, 0)),
            pl.BlockSpec((r, D), lambda i: (i, 0)),
            pl.BlockSpec((D, D), lambda i: (0, 0)),
        ],
        out_specs=pl.BlockSpec((r,), lambda i: (i,)),
        out_shape=jax.ShapeDtypeStruct((B,), jnp.float32),
        interpret=interpret,
    )(e1, e2, circ)
    return out


def kernel(source1, source2, dummy_vector, word1, word2):
    i1 = word1.astype(jnp.int32)
    i2 = word2.astype(jnp.int32)
    e1, e2 = _sc_gather(source1, source2, i1, i2)
    shift = (jnp.arange(D)[None, :] - jnp.arange(D)[:, None]) % D
    circ = dummy_vector[shift]
    return _bind_cosine(e1, e2, circ)


# speedup vs baseline: 2.6411x; 1.1960x over previous
import functools

import jax
import jax.numpy as jnp
from jax import lax
from jax.experimental import pallas as pl
from jax.experimental.pallas import tpu as pltpu
from jax.experimental.pallas import tpu_sc as plsc

D = 64
B = 16384

_ROWS_PER_BLOCK = 2048


def _sc_gather(table, idx1, idx2):
    info = plsc.get_sparse_core_info()
    nc, ns = info.num_cores, info.num_subcores
    nw = nc * ns
    b_per_w = B // nw
    mesh = plsc.VectorSubcoreMesh(core_axis_name="c", subcore_axis_name="s")

    @functools.partial(
        pl.kernel,
        mesh=mesh,
        compiler_params=pltpu.CompilerParams(use_tc_tiling_on_sc=True),
        out_type=(
            jax.ShapeDtypeStruct((B, 2 * D), jnp.float32),
            jax.ShapeDtypeStruct((B, 2 * D), jnp.float32),
        ),
        scratch_types=[
            pltpu.VMEM((b_per_w,), jnp.int32),
            pltpu.VMEM((b_per_w,), jnp.int32),
            pltpu.VMEM((b_per_w, 2 * D), jnp.float32),
            pltpu.SemaphoreType.DMA,
        ],
    )
    def gather_kernel(t, i1, i2, o1, o2, iv1, iv2, rows, sem):
        wid = lax.axis_index("s") * nc + lax.axis_index("c")
        base = wid * b_per_w
        pltpu.sync_copy(i1.at[pl.ds(base, b_per_w)], iv1)
        pltpu.sync_copy(i2.at[pl.ds(base, b_per_w)], iv2)
        pltpu.async_copy(t.at[iv1], rows, sem).wait()
        pltpu.sync_copy(rows, o1.at[pl.ds(base, b_per_w)])
        pltpu.async_copy(t.at[iv2], rows, sem).wait()
        pltpu.sync_copy(rows, o2.at[pl.ds(base, b_per_w)])

    return gather_kernel(table, idx1, idx2)


def _bind_cosine_body(g1_ref, g2_ref, c_ref, out_ref):
    c = c_ref[...]
    e1 = g1_ref[:, :D]
    e2 = g2_ref[:, D:]
    w1 = jnp.dot(e1, c, preferred_element_type=jnp.float32)
    w2 = jnp.dot(e2, c, preferred_element_type=jnp.float32)
    num = jnp.sum(w1 * w2, axis=-1)
    n1 = jnp.sum(w1 * w1, axis=-1)
    n2 = jnp.sum(w2 * w2, axis=-1)
    out_ref[...] = num / (jnp.sqrt(n1) * jnp.sqrt(n2) + 1e-8)


def _bind_cosine(g1, g2, circ, interpret=False):
    r = _ROWS_PER_BLOCK
    g = B // r
    out = pl.pallas_call(
        _bind_cosine_body,
        grid=(g,),
        in_specs=[
            pl.BlockSpec((r, 2 * D), lambda i: (i, 0)),
            pl.BlockSpec((r, 2 * D), lambda i: (i, 0)),
            pl.BlockSpec((D, D), lambda i: (0, 0)),
        ],
        out_specs=pl.BlockSpec((r,), lambda i: (i,)),
        out_shape=jax.ShapeDtypeStruct((B,), jnp.float32),
        interpret=interpret,
    )(g1, g2, circ)
    return out


def kernel(source1, source2, dummy_vector, word1, word2):
    i1 = word1.astype(jnp.int32)
    i2 = word2.astype(jnp.int32)
    table = jnp.concatenate([source1, source2], axis=1)
    g1, g2 = _sc_gather(table, i1, i2)
    shift = (jnp.arange(D)[None, :] - jnp.arange(D)[:, None]) % D
    circ = dummy_vector[shift]
    return _bind_cosine(g1, g2, circ)
